# baseline (device time: 51544 ns/iter reference)
import jax
import jax.numpy as jnp
from jax import lax
from jax.experimental import pallas as pl
from jax.experimental.pallas import tpu as pltpu

N_DEV = 32
E_PER = 4
N_BLK = 4


def kernel(x, router_W, route_idx, expert_W):
    n_tok, d_model = x.shape
    e_per, _, d_out = expert_W.shape
    chunk = n_tok // N_DEV
    blk = n_tok // N_BLK
    cpb = N_DEV // N_BLK

    def quantize(v):
        absmax = jnp.max(jnp.abs(v), axis=1, keepdims=True)
        scale = jnp.maximum(absmax, 1e-30) / 127.0
        q = jnp.clip(jnp.round(v / scale), -127, 127).astype(jnp.int8)
        return q, scale

    def body(x_ref, rw_ref, idx_ref, w_ref, out_ref,
             x16_ref, w16_ref, stageq_ref, sscale_ref,
             comm_ref, rscale_ref, gatherq_ref, gscale_ref, red_ref,
             send1_sems, send2_sems, recv1_sems, recv2_sems):
        my = lax.axis_index("i")

        barrier_sem = pltpu.get_barrier_semaphore()
        for k in range(1, N_DEV):
            peer = lax.rem(my + k, N_DEV)
            pl.semaphore_signal(
                barrier_sem, inc=1,
                device_id=(peer,), device_id_type=pl.DeviceIdType.MESH,
            )
        pl.semaphore_wait(barrier_sem, N_DEV - 1)

        x16_ref[:, :] = x_ref[:, :].astype(jnp.bfloat16)
        w16_ref[...] = w_ref[...].astype(jnp.bfloat16)

        rot = lax.rem(my, N_BLK)
        for b in range(N_BLK):
            bb = lax.rem(rot + b, N_BLK)
            rows = pl.ds(bb * blk, blk)
            xb = x16_ref[rows, :]
            mb = idx_ref[rows, :]
            acc = jnp.zeros((blk, d_out), jnp.float32)
            for j in range(E_PER):
                e = my * E_PER + j
                mask = (mb == e).astype(jnp.bfloat16)
                acc = acc + jnp.dot(
                    xb * mask, w16_ref[j], preferred_element_type=jnp.float32,
                )
            q, scale = quantize(acc)
            stageq_ref[rows, :] = q
            for i in range(cpb):
                c = bb * cpb + i
                sc = scale[i * chunk:(i + 1) * chunk, 0].reshape(1, chunk)
                sscale_ref[pl.ds(c, 1), 0:chunk] = sc
                data = pltpu.make_async_remote_copy(
                    src_ref=stageq_ref.at[pl.ds(c * chunk, chunk), :],
                    dst_ref=comm_ref.at[my],
                    send_sem=send1_sems.at[c],
                    recv_sem=recv1_sems.at[my],
                    device_id=(c,),
                    device_id_type=pl.DeviceIdType.MESH,
                )
                scl = pltpu.make_async_remote_copy(
                    src_ref=sscale_ref.at[pl.ds(c, 1), :],
                    dst_ref=rscale_ref.at[pl.ds(my, 1), :],
                    send_sem=send1_sems.at[c],
                    recv_sem=recv1_sems.at[my],
                    device_id=(c,),
                    device_id_type=pl.DeviceIdType.MESH,
                )

                @pl.when(my != c)
                def _(data=data, scl=scl):
                    data.start()
                    scl.start()

        own_sc = sscale_ref[pl.ds(my, 1), 0:chunk].reshape(chunk, 1)
        red_ref[...] = (
            stageq_ref[pl.ds(my * chunk, chunk), :].astype(jnp.float32) * own_sc
        )

        def p1_recv(k, _):
            s = lax.rem(my + 1 + k, N_DEV)
            data = pltpu.make_async_remote_copy(
                src_ref=comm_ref.at[s], dst_ref=comm_ref.at[s],
                send_sem=send1_sems.at[s], recv_sem=recv1_sems.at[s],
                device_id=(s,), device_id_type=pl.DeviceIdType.MESH,
            )
            scl = pltpu.make_async_remote_copy(
                src_ref=rscale_ref.at[pl.ds(s, 1), :],
                dst_ref=rscale_ref.at[pl.ds(s, 1), :],
                send_sem=send1_sems.at[s], recv_sem=recv1_sems.at[s],
                device_id=(s,), device_id_type=pl.DeviceIdType.MESH,
            )
            data.wait_recv()
            scl.wait_recv()
            sc = rscale_ref[pl.ds(s, 1), 0:chunk].reshape(chunk, 1)
            red_ref[...] = red_ref[...] + comm_ref[s].astype(jnp.float32) * sc
            return 0

        lax.fori_loop(0, N_DEV - 1, p1_recv, 0)

        q2, scale2 = quantize(red_ref[...])
        gatherq_ref[pl.ds(my, 1)] = q2.reshape(1, chunk, d_out)
        gscale_ref[pl.ds(my, 1), 0:chunk] = scale2.reshape(1, chunk)
        out_ref[pl.ds(my * chunk, chunk), :] = red_ref[...]

        def p2_send(k, _):
            t = lax.rem(my + 1 + k, N_DEV)
            data = pltpu.make_async_remote_copy(
                src_ref=gatherq_ref.at[my], dst_ref=gatherq_ref.at[my],
                send_sem=send2_sems.at[t], recv_sem=recv2_sems.at[my],
                device_id=(t,), device_id_type=pl.DeviceIdType.MESH,
            )
            scl = pltpu.make_async_remote_copy(
                src_ref=gscale_ref.at[pl.ds(my, 1), :],
                dst_ref=gscale_ref.at[pl.ds(my, 1), :],
                send_sem=send2_sems.at[t], recv_sem=recv2_sems.at[my],
                device_id=(t,), device_id_type=pl.DeviceIdType.MESH,
            )
            data.start()
            scl.start()
            return 0

        lax.fori_loop(0, N_DEV - 1, p2_send, 0)

        def p2_recv(k, _):
            s = lax.rem(my + 1 + k, N_DEV)
            data = pltpu.make_async_remote_copy(
                src_ref=gatherq_ref.at[s], dst_ref=gatherq_ref.at[s],
                send_sem=send2_sems.at[s], recv_sem=recv2_sems.at[s],
                device_id=(s,), device_id_type=pl.DeviceIdType.MESH,
            )
            scl = pltpu.make_async_remote_copy(
                src_ref=gscale_ref.at[pl.ds(s, 1), :],
                dst_ref=gscale_ref.at[pl.ds(s, 1), :],
                send_sem=send2_sems.at[s], recv_sem=recv2_sems.at[s],
                device_id=(s,), device_id_type=pl.DeviceIdType.MESH,
            )
            data.wait_recv()
            scl.wait_recv()
            sc = gscale_ref[pl.ds(s, 1), 0:chunk].reshape(chunk, 1)
            out_ref[pl.ds(s * chunk, chunk), :] = (
                gatherq_ref[s].astype(jnp.float32) * sc
            )
            return 0

        lax.fori_loop(0, N_DEV - 1, p2_recv, 0)

        def retire(k, _):
            t = lax.rem(my + 1 + k, N_DEV)
            for sems, dsrc, ssrc in (
                (send1_sems, stageq_ref.at[pl.ds(0, chunk), :],
                 sscale_ref.at[pl.ds(0, 1), :]),
                (send2_sems, gatherq_ref.at[0],
                 gscale_ref.at[pl.ds(0, 1), :]),
            ):
                for src in (dsrc, ssrc):
                    d = pltpu.make_async_remote_copy(
                        src_ref=src, dst_ref=src,
                        send_sem=sems.at[t], recv_sem=recv1_sems.at[t],
                        device_id=(t,), device_id_type=pl.DeviceIdType.MESH,
                    )
                    d.wait_send()
            return 0

        lax.fori_loop(0, N_DEV - 1, retire, 0)

    return pl.pallas_call(
        body,
        out_shape=jax.ShapeDtypeStruct((n_tok, d_out), jnp.float32),
        in_specs=[
            pl.BlockSpec(memory_space=pltpu.VMEM),
            pl.BlockSpec(memory_space=pltpu.VMEM),
            pl.BlockSpec(memory_space=pltpu.VMEM),
            pl.BlockSpec(memory_space=pltpu.VMEM),
        ],
        out_specs=pl.BlockSpec(memory_space=pltpu.VMEM),
        scratch_shapes=[
            pltpu.VMEM((n_tok, d_model), jnp.bfloat16),
            pltpu.VMEM((e_per, d_model, d_out), jnp.bfloat16),
            pltpu.VMEM((n_tok, d_out), jnp.int8),
            pltpu.VMEM((N_DEV, 128), jnp.float32),
            pltpu.VMEM((N_DEV, chunk, d_out), jnp.int8),
            pltpu.VMEM((N_DEV, 128), jnp.float32),
            pltpu.VMEM((N_DEV, chunk, d_out), jnp.int8),
            pltpu.VMEM((N_DEV, 128), jnp.float32),
            pltpu.VMEM((chunk, d_out), jnp.float32),
            pltpu.SemaphoreType.DMA((N_DEV,)),
            pltpu.SemaphoreType.DMA((N_DEV,)),
            pltpu.SemaphoreType.DMA((N_DEV,)),
            pltpu.SemaphoreType.DMA((N_DEV,)),
        ],
        compiler_params=pltpu.CompilerParams(collective_id=0),
    )(x, router_W, route_idx, expert_W)


# device time: 49877 ns/iter; 1.0334x vs baseline; 1.0334x over previous
import jax
import jax.numpy as jnp
from jax import lax
from jax.experimental import pallas as pl
from jax.experimental.pallas import tpu as pltpu

N_DEV = 32
E_PER = 4
N_BLK = 4


def kernel(x, router_W, route_idx, expert_W):
    n_tok, d_model = x.shape
    e_per, _, d_out = expert_W.shape
    chunk = n_tok // N_DEV
    blk = n_tok // N_BLK
    cpb = N_DEV // N_BLK

    def quantize(v):
        absmax = jnp.max(jnp.abs(v), axis=1, keepdims=True)
        scale = jnp.maximum(absmax, 1e-30) / 127.0
        q = jnp.clip(jnp.round(v / scale), -127, 127).astype(jnp.int8)
        return q, scale

    def body(x_ref, rw_ref, idx_ref, w_ref, out_ref,
             x16_ref, w16_ref, stageq_ref, sscale_ref,
             comm_ref, rscale_ref, gatherq_ref, gscale_ref, red_ref,
             send1_sems, send2_sems, recv1_sems, recv2_sems):
        my = lax.axis_index("i")

        barrier_sem = pltpu.get_barrier_semaphore()
        for k in range(1, N_DEV):
            peer = lax.rem(my + k, N_DEV)
            pl.semaphore_signal(
                barrier_sem, inc=1,
                device_id=(peer,), device_id_type=pl.DeviceIdType.MESH,
            )
        pl.semaphore_wait(barrier_sem, N_DEV - 1)

        x16_ref[:, :] = x_ref[:, :].astype(jnp.bfloat16)
        w16_ref[...] = w_ref[...].astype(jnp.bfloat16)

        rot = lax.rem(my, N_BLK)
        for b in range(N_BLK):
            bb = lax.rem(rot + b, N_BLK)
            rows = pl.ds(bb * blk, blk)
            xb = x16_ref[rows, :]
            mb = idx_ref[rows, :]
            acc = jnp.zeros((blk, d_out), jnp.float32)
            for j in range(E_PER):
                e = my * E_PER + j
                mask = (mb == e).astype(jnp.bfloat16)
                acc = acc + jnp.dot(
                    xb * mask, w16_ref[j], preferred_element_type=jnp.float32,
                )
            q, scale = quantize(acc)
            stageq_ref[rows, :] = q
            for i in range(cpb):
                c = bb * cpb + i
                sc = scale[i * chunk:(i + 1) * chunk, 0].reshape(1, chunk)
                sscale_ref[pl.ds(c, 1), 0:chunk] = sc
                data = pltpu.make_async_remote_copy(
                    src_ref=stageq_ref.at[pl.ds(c * chunk, chunk), :],
                    dst_ref=comm_ref.at[my],
                    send_sem=send1_sems.at[c],
                    recv_sem=recv1_sems.at[my],
                    device_id=(c,),
                    device_id_type=pl.DeviceIdType.MESH,
                )
                scl = pltpu.make_async_remote_copy(
                    src_ref=sscale_ref.at[pl.ds(c, 1), :],
                    dst_ref=rscale_ref.at[pl.ds(my, 1), :],
                    send_sem=send1_sems.at[c],
                    recv_sem=recv1_sems.at[my],
                    device_id=(c,),
                    device_id_type=pl.DeviceIdType.MESH,
                )

                @pl.when(my != c)
                def _(data=data, scl=scl):
                    data.start()
                    scl.start()

        own_sc = sscale_ref[pl.ds(my, 1), 0:chunk].reshape(chunk, 1)
        red_ref[...] = (
            stageq_ref[pl.ds(my * chunk, chunk), :].astype(jnp.float32) * own_sc
        )

        def p1_recv(k, _):
            s = lax.rem(my + 1 + k, N_DEV)
            data = pltpu.make_async_remote_copy(
                src_ref=comm_ref.at[s], dst_ref=comm_ref.at[s],
                send_sem=send1_sems.at[s], recv_sem=recv1_sems.at[s],
                device_id=(s,), device_id_type=pl.DeviceIdType.MESH,
            )
            scl = pltpu.make_async_remote_copy(
                src_ref=rscale_ref.at[pl.ds(s, 1), :],
                dst_ref=rscale_ref.at[pl.ds(s, 1), :],
                send_sem=send1_sems.at[s], recv_sem=recv1_sems.at[s],
                device_id=(s,), device_id_type=pl.DeviceIdType.MESH,
            )
            data.wait_recv()
            scl.wait_recv()
            sc = rscale_ref[pl.ds(s, 1), 0:chunk].reshape(chunk, 1)
            red_ref[...] = red_ref[...] + comm_ref[s].astype(jnp.float32) * sc
            return 0

        lax.fori_loop(0, N_DEV - 1, p1_recv, 0)

        q2, scale2 = quantize(red_ref[...])
        gatherq_ref[pl.ds(my, 1)] = q2.reshape(1, chunk, d_out)
        gscale_ref[pl.ds(my, 1), 0:chunk] = scale2.reshape(1, chunk)

        def p2_send(k, _):
            t = lax.rem(my + 1 + k, N_DEV)
            data = pltpu.make_async_remote_copy(
                src_ref=gatherq_ref.at[my], dst_ref=gatherq_ref.at[my],
                send_sem=send2_sems.at[t], recv_sem=recv2_sems.at[my],
                device_id=(t,), device_id_type=pl.DeviceIdType.MESH,
            )
            scl = pltpu.make_async_remote_copy(
                src_ref=gscale_ref.at[pl.ds(my, 1), :],
                dst_ref=gscale_ref.at[pl.ds(my, 1), :],
                send_sem=send2_sems.at[t], recv_sem=recv2_sems.at[my],
                device_id=(t,), device_id_type=pl.DeviceIdType.MESH,
            )
            data.start()
            scl.start()
            return 0

        lax.fori_loop(0, N_DEV - 1, p2_send, 0)

        def p2_recv(k, _):
            s = lax.rem(my + 1 + k, N_DEV)
            data = pltpu.make_async_remote_copy(
                src_ref=gatherq_ref.at[s], dst_ref=gatherq_ref.at[s],
                send_sem=send2_sems.at[s], recv_sem=recv2_sems.at[s],
                device_id=(s,), device_id_type=pl.DeviceIdType.MESH,
            )
            scl = pltpu.make_async_remote_copy(
                src_ref=gscale_ref.at[pl.ds(s, 1), :],
                dst_ref=gscale_ref.at[pl.ds(s, 1), :],
                send_sem=send2_sems.at[s], recv_sem=recv2_sems.at[s],
                device_id=(s,), device_id_type=pl.DeviceIdType.MESH,
            )
            data.wait_recv()
            scl.wait_recv()
            return 0

        lax.fori_loop(0, N_DEV - 1, p2_recv, 0)

        gsc = gscale_ref[:, 0:chunk].reshape(N_DEV, chunk, 1)
        out_ref[:, :] = (
            gatherq_ref[...].astype(jnp.float32) * gsc
        ).reshape(n_tok, d_out)

        def retire(k, _):
            t = lax.rem(my + 1 + k, N_DEV)
            for sems, dsrc, ssrc in (
                (send1_sems, stageq_ref.at[pl.ds(0, chunk), :],
                 sscale_ref.at[pl.ds(0, 1), :]),
                (send2_sems, gatherq_ref.at[0],
                 gscale_ref.at[pl.ds(0, 1), :]),
            ):
                for src in (dsrc, ssrc):
                    d = pltpu.make_async_remote_copy(
                        src_ref=src, dst_ref=src,
                        send_sem=sems.at[t], recv_sem=recv1_sems.at[t],
                        device_id=(t,), device_id_type=pl.DeviceIdType.MESH,
                    )
                    d.wait_send()
            return 0

        lax.fori_loop(0, N_DEV - 1, retire, 0)

    return pl.pallas_call(
        body,
        out_shape=jax.ShapeDtypeStruct((n_tok, d_out), jnp.float32),
        in_specs=[
            pl.BlockSpec(memory_space=pltpu.VMEM),
            pl.BlockSpec(memory_space=pltpu.VMEM),
            pl.BlockSpec(memory_space=pltpu.VMEM),
            pl.BlockSpec(memory_space=pltpu.VMEM),
        ],
        out_specs=pl.BlockSpec(memory_space=pltpu.VMEM),
        scratch_shapes=[
            pltpu.VMEM((n_tok, d_model), jnp.bfloat16),
            pltpu.VMEM((e_per, d_model, d_out), jnp.bfloat16),
            pltpu.VMEM((n_tok, d_out), jnp.int8),
            pltpu.VMEM((N_DEV, 128), jnp.float32),
            pltpu.VMEM((N_DEV, chunk, d_out), jnp.int8),
            pltpu.VMEM((N_DEV, 128), jnp.float32),
            pltpu.VMEM((N_DEV, chunk, d_out), jnp.int8),
            pltpu.VMEM((N_DEV, 128), jnp.float32),
            pltpu.VMEM((chunk, d_out), jnp.float32),
            pltpu.SemaphoreType.DMA((N_DEV,)),
            pltpu.SemaphoreType.DMA((N_DEV,)),
            pltpu.SemaphoreType.DMA((N_DEV,)),
            pltpu.SemaphoreType.DMA((N_DEV,)),
        ],
        compiler_params=pltpu.CompilerParams(collective_id=0),
    )(x, router_W, route_idx, expert_W)
